# baseline (speedup 1.0000x reference)
import functools

import jax
import jax.numpy as jnp
from jax import lax
from jax.experimental import pallas as pl
from jax.experimental.pallas import tpu as pltpu
from jax.experimental.pallas import tpu_sc as plsc

B, L, D = 16384, 50, 32
LH = 25
BB = 128
NTI = D // 8

_info = plsc.get_sparse_core_info()
NC, NS = _info.num_cores, _info.num_subcores
NW = NC * NS
TASKS_PER_W = (B // BB) * 2 // NW


@functools.lru_cache(maxsize=None)
def _make_gather(vocab: int):
    mesh = plsc.VectorSubcoreMesh(core_axis_name="c", subcore_axis_name="s")

    @functools.partial(
        pl.kernel,
        mesh=mesh,
        out_type=jax.ShapeDtypeStruct((L, NTI, B // BB, 8, BB), jnp.float32),
        scratch_types=[
            pltpu.VMEM((BB, 2, 32), jnp.int32),
            pltpu.VMEM((BB * LH, D), jnp.float32),
            pltpu.VMEM((2, NTI, 8, BB), jnp.float32),
            pltpu.SemaphoreType.DMA,
            pltpu.SemaphoreType.DMA,
            pltpu.SemaphoreType.DMA,
        ],
        compiler_params=pltpu.CompilerParams(use_tc_tiling_on_sc=False, needs_layout_passes=False),
    )
    def body(idx_hbm, table_hbm, out_hbm, idx_v, buf, tiles, gsem, ws0, ws1):
        wid = lax.axis_index("s") * NC + lax.axis_index("c")
        wsem = (ws0, ws1)
        lane = lax.iota(jnp.int32, 16)
        rowbase = [(lane + 16 * j) * LH for j in range(BB // 16)]

        def transpose_l(l_local, parity, dst_l, tj):
            for ti in range(NTI):
                for r in range(8):
                    col = jnp.full((16,), 8 * ti + r, jnp.int32)
                    for j in range(BB // 16):
                        v = plsc.load_gather(
                            buf, [rowbase[j] + l_local, col])
                        tiles[parity, ti, r, pl.ds(16 * j, 16)] = v
            for ti in range(NTI):
                pltpu.async_copy(
                    tiles.at[parity].at[ti],
                    out_hbm.at[dst_l].at[ti].at[tj],
                    wsem[parity])

        def wait_tiles(parity):
            for _ in range(NTI):
                pltpu.make_async_copy(
                    out_hbm.at[0].at[0].at[0],
                    tiles.at[parity].at[0],
                    wsem[parity]).wait()

        def task(k, carry):
            t = wid * TASKS_PER_W + k
            tj = t // 2
            lh = t % 2
            l0 = lh * LH
            pltpu.sync_copy(idx_hbm.at[pl.ds(tj * BB, BB)], idx_v)

            def fire(b, c2):
                pltpu.async_copy(
                    table_hbm.at[idx_v.at[b].at[lh].at[pl.ds(0, LH)]],
                    buf.at[pl.ds(b * LH, LH)],
                    gsem)
                return c2

            lax.fori_loop(0, BB, fire, 0)
            pltpu.make_async_copy(
                table_hbm.at[pl.ds(0, BB * LH)], buf, gsem).wait()

            @pl.when(k > 0)
            def _():
                wait_tiles(0)
                wait_tiles(1)
            transpose_l(0, 0, l0, tj)
            transpose_l(1, 1, l0 + 1, tj)

            def pair(q, c2):
                l = 2 * q
                wait_tiles(0)
                transpose_l(l, 0, l0 + l, tj)
                wait_tiles(1)
                transpose_l(l + 1, 1, l0 + l + 1, tj)
                return c2

            lax.fori_loop(1, LH // 2, pair, 0)
            wait_tiles(0)
            transpose_l(LH - 1, 0, l0 + LH - 1, tj)
            return carry

        lax.fori_loop(0, TASKS_PER_W, task, 0)
        wait_tiles(0)
        wait_tiles(1)

    return body


def kernel(indices, table):
    idx_p = jnp.pad(indices.reshape(B, 2, LH), ((0, 0), (0, 0), (0, 32 - LH)))
    o5 = _make_gather(table.shape[0])(idx_p, table)
    return jnp.transpose(o5, (2, 4, 0, 1, 3)).reshape(B, L, D)


# speedup vs baseline: 1.4275x; 1.4275x over previous
import functools

import jax
import jax.numpy as jnp
from jax import lax
from jax.experimental import pallas as pl
from jax.experimental.pallas import tpu as pltpu
from jax.experimental.pallas import tpu_sc as plsc

B, L, D = 16384, 50, 32
LH = 25
BB = 128
NTI = D // 8

_info = plsc.get_sparse_core_info()
NC, NS = _info.num_cores, _info.num_subcores
NW = NC * NS
TASKS_PER_W = (B // BB) * 2 // NW


@functools.lru_cache(maxsize=None)
def _make_gather(vocab: int):
    mesh = plsc.VectorSubcoreMesh(core_axis_name="c", subcore_axis_name="s")

    @functools.partial(
        pl.kernel,
        mesh=mesh,
        out_type=jax.ShapeDtypeStruct((L, NTI, B // BB, 8, BB), jnp.float32),
        scratch_types=[
            pltpu.VMEM((BB, 2, 32), jnp.int32),
            pltpu.VMEM((BB * LH, D), jnp.float32),
            pltpu.VMEM((2, D, 137), jnp.float32),
            pltpu.SemaphoreType.DMA,
            pltpu.SemaphoreType.DMA,
            pltpu.SemaphoreType.DMA,
        ],
        compiler_params=pltpu.CompilerParams(use_tc_tiling_on_sc=False, needs_layout_passes=False),
    )
    def body(idx_hbm, table_hbm, out_hbm, idx_v, buf, tiles, gsem, ws0, ws1):
        wid = lax.axis_index("s") * NC + lax.axis_index("c")
        wsem = (ws0, ws1)
        lane = lax.iota(jnp.int32, 16)
        lane16 = lane + 16

        def transpose_l(l_local, parity, dst_l, tj):
            for b in range(BB):
                row = b * LH + l_local
                v0 = buf[row, pl.ds(0, 16)]
                v1 = buf[row, pl.ds(16, 16)]
                cb = jnp.full((16,), b, jnp.int32)
                plsc.store_scatter(tiles.at[parity], [lane, cb], v0)
                plsc.store_scatter(tiles.at[parity], [lane16, cb], v1)
            for ti in range(NTI):
                pltpu.async_copy(
                    tiles.at[parity].at[pl.ds(8 * ti, 8)].at[:, pl.ds(0, BB)],
                    out_hbm.at[dst_l].at[ti].at[tj],
                    wsem[parity])

        def wait_tiles(parity):
            for _ in range(NTI):
                pltpu.make_async_copy(
                    out_hbm.at[0].at[0].at[0],
                    tiles.at[parity].at[pl.ds(0, 8)].at[:, pl.ds(0, BB)],
                    wsem[parity]).wait()

        def task(k, carry):
            t = wid * TASKS_PER_W + k
            tj = t // 2
            lh = t % 2
            l0 = lh * LH
            pltpu.sync_copy(idx_hbm.at[pl.ds(tj * BB, BB)], idx_v)

            def fire(b, c2):
                pltpu.async_copy(
                    table_hbm.at[idx_v.at[b].at[lh].at[pl.ds(0, LH)]],
                    buf.at[pl.ds(b * LH, LH)],
                    gsem)
                return c2

            lax.fori_loop(0, BB, fire, 0)
            pltpu.make_async_copy(
                table_hbm.at[pl.ds(0, BB * LH)], buf, gsem).wait()

            @pl.when(k > 0)
            def _():
                wait_tiles(0)
                wait_tiles(1)
            transpose_l(0, 0, l0, tj)
            transpose_l(1, 1, l0 + 1, tj)

            def pair(q, c2):
                l = 2 * q
                wait_tiles(0)
                transpose_l(l, 0, l0 + l, tj)
                wait_tiles(1)
                transpose_l(l + 1, 1, l0 + l + 1, tj)
                return c2

            lax.fori_loop(1, LH // 2, pair, 0)
            wait_tiles(0)
            transpose_l(LH - 1, 0, l0 + LH - 1, tj)
            return carry

        lax.fori_loop(0, TASKS_PER_W, task, 0)
        wait_tiles(0)
        wait_tiles(1)

    return body


def kernel(indices, table):
    idx_p = jnp.pad(indices.reshape(B, 2, LH), ((0, 0), (0, 0), (0, 32 - LH)))
    o5 = _make_gather(table.shape[0])(idx_p, table)
    return jnp.transpose(o5, (2, 4, 0, 1, 3)).reshape(B, L, D)


# speedup vs baseline: 1.4648x; 1.0261x over previous
import functools

import jax
import jax.numpy as jnp
from jax import lax
from jax.experimental import pallas as pl
from jax.experimental.pallas import tpu as pltpu
from jax.experimental.pallas import tpu_sc as plsc

B, L, D = 16384, 50, 32
LH = 25
LA, LB = 16, 9
BB = 128
NTI = D // 8
TS = 137

_info = plsc.get_sparse_core_info()
NC, NS = _info.num_cores, _info.num_subcores
NW = NC * NS
TASKS_PER_W = (B // BB) * 2 // NW


@functools.lru_cache(maxsize=None)
def _make_gather(vocab: int):
    mesh = plsc.VectorSubcoreMesh(core_axis_name="c", subcore_axis_name="s")

    @functools.partial(
        pl.kernel,
        mesh=mesh,
        out_type=jax.ShapeDtypeStruct((L, NTI, B // BB, 8, BB), jnp.float32),
        scratch_types=[
            pltpu.VMEM((BB, 2, 32), jnp.int32),
            pltpu.VMEM((BB * LA, D), jnp.float32),
            pltpu.VMEM((BB * LB, D), jnp.float32),
            pltpu.VMEM((2, D, TS), jnp.float32),
            pltpu.SemaphoreType.DMA,
            pltpu.SemaphoreType.DMA,
            pltpu.SemaphoreType.DMA,
            pltpu.SemaphoreType.DMA,
        ],
        compiler_params=pltpu.CompilerParams(
            use_tc_tiling_on_sc=False, needs_layout_passes=False),
    )
    def body(idx_hbm, table_hbm, out_hbm, idx_v, buf_a, buf_b, tiles,
             gsa, gsb, ws0, ws1):
        wid = lax.axis_index("s") * NC + lax.axis_index("c")
        wsem = (ws0, ws1)
        lane = lax.iota(jnp.int32, 16)
        lane16 = lane + 16

        def transpose_l(buf, stride, l_local, parity, dst_l, tj):
            def blk(i, c2):
                for bo in range(8):
                    b = i * 8 + bo
                    row = b * stride + l_local
                    v0 = buf[row, pl.ds(0, 16)]
                    v1 = buf[row, pl.ds(16, 16)]
                    cb = jnp.zeros((16,), jnp.int32) + b
                    plsc.store_scatter(tiles.at[parity], [lane, cb], v0)
                    plsc.store_scatter(tiles.at[parity], [lane16, cb], v1)
                return c2

            lax.fori_loop(0, BB // 8, blk, 0)
            for ti in range(NTI):
                pltpu.async_copy(
                    tiles.at[parity].at[pl.ds(8 * ti, 8)].at[:, pl.ds(0, BB)],
                    out_hbm.at[dst_l].at[ti].at[tj],
                    wsem[parity])

        def wait_tiles(parity):
            for _ in range(NTI):
                pltpu.make_async_copy(
                    out_hbm.at[0].at[0].at[0],
                    tiles.at[parity].at[pl.ds(0, 8)].at[:, pl.ds(0, BB)],
                    wsem[parity]).wait()

        def stage_idx(tj):
            pltpu.sync_copy(idx_hbm.at[pl.ds(tj * BB, BB)], idx_v)

        def fire(lh, l_off, ln, buf, sem):
            def one(b, c2):
                pltpu.async_copy(
                    table_hbm.at[idx_v.at[b].at[lh].at[pl.ds(l_off, ln)]],
                    buf.at[pl.ds(b * ln, ln)],
                    sem)
                return c2
            lax.fori_loop(0, BB, one, 0)

        def drain(n, buf, sem):
            pltpu.make_async_copy(
                table_hbm.at[pl.ds(0, n)], buf, sem).wait()

        def slab(buf, stride, nl, l0, tj, guard=None):
            for l in range(2):
                if guard is not None:
                    @pl.when(guard)
                    def _():
                        wait_tiles(l)
                else:
                    wait_tiles(l)
                transpose_l(buf, stride, l, l, l0 + l, tj)

            def pair(q, c2):
                l = 2 * q
                wait_tiles(0)
                transpose_l(buf, stride, l, 0, l0 + l, tj)
                wait_tiles(1)
                transpose_l(buf, stride, l + 1, 1, l0 + l + 1, tj)
                return c2

            lax.fori_loop(1, nl // 2, pair, 0)
            if nl % 2:
                wait_tiles(0)
                transpose_l(buf, stride, nl - 1, 0, l0 + nl - 1, tj)

        def task(k, carry):
            t = wid * TASKS_PER_W + k
            tj = t // 2
            lh = t % 2
            l0 = lh * LH
            stage_idx(tj)
            fire(lh, 0, LA, buf_a, gsa)

            @pl.when(k > 0)
            def _():
                tp = t - 1
                drain(BB * LB, buf_b, gsb)
                slab(buf_b, LB, LB, (tp % 2) * LH + LA, tp // 2)

            drain(BB * LA, buf_a, gsa)
            fire(lh, 16, LB, buf_b, gsb)
            slab(buf_a, LA, LA, l0, tj, guard=k > 0)
            return carry

        lax.fori_loop(0, TASKS_PER_W, task, 0)

        tlast = wid * TASKS_PER_W + TASKS_PER_W - 1
        drain(BB * LB, buf_b, gsb)
        slab(buf_b, LB, LB, (tlast % 2) * LH + LA, tlast // 2)
        wait_tiles(0)
        wait_tiles(1)

    return body


def kernel(indices, table):
    idx_p = jnp.pad(indices.reshape(B, 2, LH), ((0, 0), (0, 0), (0, 32 - LH)))
    o5 = _make_gather(table.shape[0])(idx_p, table)
    return jnp.transpose(o5, (2, 4, 0, 1, 3)).reshape(B, L, D)
